# baseline (speedup 1.0000x reference)
import jax
import jax.numpy as jnp
from jax import lax
from jax.experimental import pallas as pl
from jax.experimental.pallas import tpu as pltpu
from jax.experimental.pallas import tpu_sc as plsc

_VOCAB = 151552
_DIM = 1024
_BATCH = 4
_SEQ = 8192

_INFO = plsc.get_sparse_core_info()
_NC, _NS = _INFO.num_cores, _INFO.num_subcores
_NW = _NC * _NS
_N = _BATCH * _SEQ
_R = _N // _NW
_C = 8
_NBUF = 8
_LEAD = 4
_NSLOT = 4
_NCHUNK = _R // _C
_NROUND = _NCHUNK // _NBUF


def _gather_body(ids_hbm, table_hbm, out_hbm, idx_v, rows_v, shared_v, *sems):
    gsem = sems[:_NBUF]
    xsem = sems[_NBUF : _NBUF + _NSLOT]
    dsem = sems[_NBUF + _NSLOT :]
    wid = lax.axis_index("s") * _NC + lax.axis_index("c")
    sid = lax.axis_index("s")
    base = wid * _R
    pltpu.sync_copy(ids_hbm.at[pl.ds(base, _R)], idx_v)

    def start_g(g, b):
        pltpu.async_copy(
            table_hbm.at[idx_v.at[pl.ds(g * _C, _C)]], rows_v.at[b], gsem[b]
        )

    def wait_g(b):
        pltpu.make_async_copy(
            table_hbm.at[idx_v.at[pl.ds(0, _C)]], rows_v.at[b], gsem[b]
        ).wait()

    def start_x(b, s):
        pltpu.async_copy(rows_v.at[b], shared_v.at[sid, s], xsem[s])

    def wait_x(s):
        pltpu.make_async_copy(
            rows_v.at[0], shared_v.at[sid, s], xsem[s]
        ).wait()

    def start_d(g, s):
        pltpu.async_copy(
            shared_v.at[sid, s], out_hbm.at[pl.ds(base + g * _C, _C)], dsem[s]
        )

    def wait_d(s):
        pltpu.make_async_copy(
            shared_v.at[sid, s], out_hbm.at[pl.ds(base, _C)], dsem[s]
        ).wait()

    for g in range(_LEAD):
        start_g(g, g % _NBUF)
    for j in range(_NBUF):
        b, s, sp, b2 = j % _NBUF, j % _NSLOT, (j - 1) % _NSLOT, (j + _LEAD) % _NBUF
        wait_g(b)
        if j >= 1:
            wait_x(sp)
            start_d(j - 1, sp)
        if j >= _NSLOT:
            wait_d(s)
        start_x(b, s)
        start_g(j + _LEAD, b2)

    def round_body(gg):
        for j in range(_NBUF):
            g = gg * _NBUF + j
            b, s = j % _NBUF, j % _NSLOT
            sp, b2 = (j - 1) % _NSLOT, (j + _LEAD) % _NBUF
            wait_g(b)
            wait_x(sp)
            start_d(g - 1, sp)
            wait_d(s)
            start_x(b, s)
            start_g(g + _LEAD, b2)

    pl.loop(1, _NROUND - 1)(round_body)

    for j in range(_NBUF):
        g = (_NROUND - 1) * _NBUF + j
        b, s = j % _NBUF, j % _NSLOT
        sp, b2 = (j - 1) % _NSLOT, (j + _LEAD) % _NBUF
        wait_g(b)
        wait_x(sp)
        start_d(g - 1, sp)
        wait_d(s)
        start_x(b, s)
        if j < _NBUF - _LEAD:
            start_g(g + _LEAD, b2)
    last = _NCHUNK - 1
    wait_x(last % _NSLOT)
    start_d(last, last % _NSLOT)
    for s in range(_NSLOT):
        wait_d(s)


@jax.jit
def kernel(input_ids, word_embeddings):
    ids = input_ids.reshape(-1).astype(jnp.int32)
    mesh = plsc.VectorSubcoreMesh(core_axis_name="c", subcore_axis_name="s")
    out = pl.kernel(
        _gather_body,
        out_type=jax.ShapeDtypeStruct((_N, _DIM), jnp.float32),
        mesh=mesh,
        scratch_types=[
            pltpu.VMEM((_R,), jnp.int32),
            pltpu.VMEM((_NBUF, _C, _DIM), jnp.float32),
            pltpu.VMEM_SHARED((_NS, _NSLOT, _C, _DIM), jnp.float32),
        ]
        + [pltpu.SemaphoreType.DMA] * (_NBUF + 2 * _NSLOT),
    )(ids, word_embeddings)
    return out.reshape(_BATCH, _SEQ, _DIM)
